# baseline (speedup 1.0000x reference)
import jax
import jax.numpy as jnp
from jax.experimental import pallas as pl
from jax.experimental.pallas import tpu as pltpu


def _vq_body(x_ref, emb_ref, embt_ref, q_ref, enc_ref, idx_ref, loss_ref,
             idxbuf_ref):
    i = pl.program_id(0)
    nblk = pl.num_programs(0)
    rows, t, d = x_ref.shape
    x = x_ref[...].reshape(rows * t, d)
    emb = emb_ref[...]
    blk, k = enc_ref.shape

    sim = jnp.dot(x, emb, preferred_element_type=jnp.float32)
    xsq = jnp.sum(x * x, axis=1, keepdims=True)
    esq = jnp.sum(emb * emb, axis=0, keepdims=True)
    dist = xsq - 2.0 * sim + esq

    idx = jnp.argmin(dist, axis=1).astype(jnp.int32)
    iota = jax.lax.broadcasted_iota(jnp.int32, (blk, k), 1)

    onehot = (iota == idx[:, None]).astype(jnp.float32)
    enc_ref[...] = onehot
    idxbuf_ref[:, pl.ds(i * blk, blk)] = idx.reshape(1, blk)

    @pl.when(i == nblk - 1)
    def _emit_idx():
        idx_ref[...] = idxbuf_ref[...].reshape(idx_ref.shape)

    q = jnp.dot(onehot, embt_ref[...], preferred_element_type=jnp.float32)
    q_ref[...] = q.reshape(rows, t, d)

    diff = q - x
    part = jnp.sum(diff * diff).reshape(1, 1)

    @pl.when(i == 0)
    def _init():
        loss_ref[...] = jnp.zeros((1, 1), jnp.float32)
    loss_ref[...] += part


def kernel(inputs, embeddings):
    b, t, d = inputs.shape
    k = embeddings.shape[1]
    n = b * t
    blk = 2048
    rows = blk // t
    grid = n // blk

    embt = embeddings.T

    q, enc, idx, loss_acc = pl.pallas_call(
        _vq_body,
        grid=(grid,),
        in_specs=[
            pl.BlockSpec((rows, t, d), lambda i: (i, 0, 0)),
            pl.BlockSpec((d, k), lambda i: (0, 0)),
            pl.BlockSpec((k, d), lambda i: (0, 0)),
        ],
        out_specs=[
            pl.BlockSpec((rows, t, d), lambda i: (i, 0, 0)),
            pl.BlockSpec((blk, k), lambda i: (i, 0)),
            pl.BlockSpec((b, t), lambda i: (0, 0)),
            pl.BlockSpec((1, 1), lambda i: (0, 0)),
        ],
        out_shape=[
            jax.ShapeDtypeStruct((b, t, d), jnp.float32),
            jax.ShapeDtypeStruct((n, k), jnp.float32),
            jax.ShapeDtypeStruct((b, t), jnp.int32),
            jax.ShapeDtypeStruct((1, 1), jnp.float32),
        ],
        scratch_shapes=[pltpu.VMEM((1, n), jnp.int32)],
        compiler_params=pltpu.CompilerParams(
            vmem_limit_bytes=120 * 1024 * 1024,
        ),
    )(inputs, embeddings, embt)

    loss = loss_acc[0, 0] * (1.25 / (n * d))
    return q, enc, idx, loss


# speedup vs baseline: 1.1993x; 1.1993x over previous
import jax
import jax.numpy as jnp
from jax.experimental import pallas as pl
from jax.experimental.pallas import tpu as pltpu


def _vq_body(x_ref, emb_ref, embt_ref, q_ref, enc_ref, idx_ref, loss_ref):
    i = pl.program_id(0)
    rows, t, d = x_ref.shape
    x = x_ref[...].reshape(rows * t, d)
    emb = emb_ref[...]
    blk, k = enc_ref.shape

    sim = jnp.dot(x, emb, preferred_element_type=jnp.float32)
    xsq = jnp.sum(x * x, axis=1, keepdims=True)
    esq = jnp.sum(emb * emb, axis=0, keepdims=True)
    dist = xsq - 2.0 * sim + esq

    idx = jnp.argmin(dist, axis=1).astype(jnp.int32)
    iota = jax.lax.broadcasted_iota(jnp.int32, (blk, k), 1)

    onehot = (iota == idx[:, None]).astype(jnp.float32)
    enc_ref[...] = onehot
    idx_ref[...] = idx.reshape(1, blk)

    q = jnp.dot(onehot, embt_ref[...], preferred_element_type=jnp.float32)
    q_ref[...] = q.reshape(rows, t, d)

    diff = q - x
    part = jnp.sum(diff * diff).reshape(1, 1)

    @pl.when(i == 0)
    def _init():
        loss_ref[...] = jnp.zeros((1, 1), jnp.float32)
    loss_ref[...] += part


def kernel(inputs, embeddings):
    b, t, d = inputs.shape
    k = embeddings.shape[1]
    n = b * t
    blk = 2048
    rows = blk // t
    grid = n // blk

    embt = embeddings.T

    q, enc, idx, loss_acc = pl.pallas_call(
        _vq_body,
        grid=(grid,),
        in_specs=[
            pl.BlockSpec((rows, t, d), lambda i: (i, 0, 0)),
            pl.BlockSpec((d, k), lambda i: (0, 0)),
            pl.BlockSpec((k, d), lambda i: (0, 0)),
        ],
        out_specs=[
            pl.BlockSpec((rows, t, d), lambda i: (i, 0, 0)),
            pl.BlockSpec((blk, k), lambda i: (i, 0)),
            pl.BlockSpec((1, blk), lambda i: (0, i)),
            pl.BlockSpec((1, 1), lambda i: (0, 0)),
        ],
        out_shape=[
            jax.ShapeDtypeStruct((b, t, d), jnp.float32),
            jax.ShapeDtypeStruct((n, k), jnp.float32),
            jax.ShapeDtypeStruct((1, n), jnp.int32),
            jax.ShapeDtypeStruct((1, 1), jnp.float32),
        ],
        compiler_params=pltpu.CompilerParams(
            vmem_limit_bytes=120 * 1024 * 1024,
        ),
    )(inputs, embeddings, embt)

    loss = loss_acc[0, 0] * (1.25 / (n * d))
    return q, enc, idx.reshape(b, t), loss
